# baseline (speedup 1.0000x reference)
import jax
import jax.numpy as jnp
from jax.experimental import pallas as pl
from jax.experimental.pallas import tpu as pltpu


def _gcn_block_kernel(x_ref, adj_a_ref, adj_b_ref, w_ref, b_ref,
                      out_a_ref, out_b_ref, support_ref):
    @pl.when(pl.program_id(0) == 0)
    def _():
        support_ref[...] = jnp.dot(
            x_ref[...], w_ref[...], preferred_element_type=jnp.float32
        )

    s = support_ref[...]
    bias = b_ref[...]
    out_a_ref[...] = (
        jnp.dot(adj_a_ref[...], s, preferred_element_type=jnp.float32) + bias
    )
    out_b_ref[...] = (
        jnp.dot(adj_b_ref[...], s, preferred_element_type=jnp.float32) + bias
    )


def kernel(x, adj, W, b):
    n, din = x.shape
    dout = W.shape[1]
    half = n // 2
    bm = next(c for c in (256, 200, 128, 104, 64, 40, 16, 8, half) if half % c == 0)
    nblk = half // bm
    b2 = b.reshape(1, dout).astype(jnp.float32)

    out_a, out_b = pl.pallas_call(
        _gcn_block_kernel,
        grid=(nblk,),
        in_specs=[
            pl.BlockSpec((n, din), lambda i: (0, 0)),
            pl.BlockSpec((bm, n), lambda i: (i, 0)),
            pl.BlockSpec((bm, n), lambda i: (i + nblk, 0)),
            pl.BlockSpec((din, dout), lambda i: (0, 0)),
            pl.BlockSpec((1, dout), lambda i: (0, 0)),
        ],
        out_specs=[
            pl.BlockSpec((bm, dout), lambda i: (i, 0)),
            pl.BlockSpec((bm, dout), lambda i: (i, 0)),
        ],
        out_shape=[
            jax.ShapeDtypeStruct((half, dout), jnp.float32),
            jax.ShapeDtypeStruct((half, dout), jnp.float32),
        ],
        scratch_shapes=[pltpu.VMEM((n, dout), jnp.float32)],
    )(x, adj, adj, W, b2)
    return jnp.concatenate([out_a, out_b], axis=0)


# speedup vs baseline: 1.0100x; 1.0100x over previous
import jax
import jax.numpy as jnp
from jax.experimental import pallas as pl
from jax.experimental.pallas import tpu as pltpu


def _support_kernel(x_ref, w_ref, out_ref):
    out_ref[...] = jnp.dot(x_ref[...], w_ref[...], preferred_element_type=jnp.float32)


def _agg_kernel(support_ref, adj_ref, b_ref, out_ref):
    out_ref[...] = (
        jnp.dot(adj_ref[...], support_ref[...], preferred_element_type=jnp.float32)
        + b_ref[...]
    )


def _pick_block_rows(n: int) -> int:
    for bm in (512, 400, 256, 200, 128, 80, 40, 16, 8):
        if n % bm == 0:
            return bm
    return n


def kernel(x, adj, W, b):
    n, din = x.shape
    dout = W.shape[1]
    bm = _pick_block_rows(n)
    b2 = b.reshape(1, dout).astype(jnp.float32)

    support = pl.pallas_call(
        _support_kernel,
        out_shape=jax.ShapeDtypeStruct((n, dout), jnp.float32),
    )(x, W)

    return pl.pallas_call(
        _agg_kernel,
        grid=(n // bm,),
        in_specs=[
            pl.BlockSpec((n, dout), lambda i: (0, 0)),
            pl.BlockSpec((bm, n), lambda i: (i, 0)),
            pl.BlockSpec((1, dout), lambda i: (0, 0)),
        ],
        out_specs=pl.BlockSpec((bm, dout), lambda i: (i, 0)),
        out_shape=jax.ShapeDtypeStruct((n, dout), jnp.float32),
        compiler_params=pltpu.CompilerParams(
            dimension_semantics=("parallel",),
        ),
    )(support, adj, b2)


# speedup vs baseline: 1.0523x; 1.0418x over previous
import jax
import jax.numpy as jnp
from jax.experimental import pallas as pl
from jax.experimental.pallas import tpu as pltpu


def _gcn_block_kernel(x_ref, adj_ref, w_ref, b_ref, out_ref, support_ref):
    @pl.when(pl.program_id(0) == 0)
    def _():
        support_ref[...] = jnp.dot(
            x_ref[...], w_ref[...], preferred_element_type=jnp.float32
        )

    out_ref[...] = (
        jnp.dot(adj_ref[...], support_ref[...], preferred_element_type=jnp.float32)
        + b_ref[...]
    )


def _pick_block_rows(n: int) -> int:
    for bm in (512, 400, 256, 200, 128, 80, 40, 16, 8):
        if n % bm == 0:
            return bm
    return n


def kernel(x, adj, W, b):
    n, din = x.shape
    dout = W.shape[1]
    bm = _pick_block_rows(n)
    b2 = b.reshape(1, dout).astype(jnp.float32)

    return pl.pallas_call(
        _gcn_block_kernel,
        grid=(n // bm,),
        in_specs=[
            pl.BlockSpec((n, din), lambda i: (0, 0)),
            pl.BlockSpec((bm, n), lambda i: (i, 0)),
            pl.BlockSpec((din, dout), lambda i: (0, 0)),
            pl.BlockSpec((1, dout), lambda i: (0, 0)),
        ],
        out_specs=pl.BlockSpec((bm, dout), lambda i: (i, 0)),
        out_shape=jax.ShapeDtypeStruct((n, dout), jnp.float32),
        scratch_shapes=[pltpu.VMEM((n, dout), jnp.float32)],
    )(x, adj, W, b2)
